# Initial kernel scaffold; baseline (speedup 1.0000x reference)
import functools

import jax
import jax.numpy as jnp
from jax import lax
from jax.experimental import pallas as pl
from jax.experimental.pallas import tpu as pltpu
from jax.experimental.pallas import tpu_sc as plsc

VOCAB = 100000
EMBED = 32
BATCH = 16384
SEQ = 200

NC, NS = 2, 16
NW = NC * NS
NROWS = BATCH * SEQ
IW = 128
NIDXROWS = NROWS // IW
IDXROWS_PER_W = NIDXROWS // NW
CH = 8
ITERS = IDXROWS_PER_W // CH


@functools.partial(
    pl.kernel,
    out_type=jax.ShapeDtypeStruct((NIDXROWS, IW, EMBED), jnp.float32),
    mesh=plsc.VectorSubcoreMesh(core_axis_name="c", subcore_axis_name="s"),
    scratch_types=[
        pltpu.VMEM((CH, IW), jnp.int32),
        pltpu.VMEM((CH, IW, EMBED), jnp.float32),
        pltpu.SemaphoreType.DMA,
    ],
)
def _gather_all(tbl_hbm, idx_hbm, out_hbm, idx_v, rows_v, sem):
    wid = lax.axis_index("s") * NC + lax.axis_index("c")
    row0 = wid * IDXROWS_PER_W

    def body(g, carry):
        r0 = row0 + g * CH
        pltpu.sync_copy(idx_hbm.at[pl.ds(r0, CH)], idx_v)
        copies = [
            pltpu.async_copy(tbl_hbm.at[idx_v.at[j]], rows_v.at[j], sem)
            for j in range(CH)
        ]
        for c in copies:
            c.wait()
        pltpu.sync_copy(rows_v, out_hbm.at[pl.ds(r0, CH)])
        return carry

    lax.fori_loop(0, ITERS, body, 0)


def kernel(inchi, table, start_var):
    b, l = inchi.shape
    tok = inchi[:, :-1].astype(jnp.int32)
    idx = jnp.concatenate(
        [jnp.full((b, 1), VOCAB, jnp.int32), tok], axis=1)
    idx_rows = idx.reshape(NIDXROWS, IW)
    tbl = jnp.concatenate([table, start_var], axis=0)
    out = _gather_all(tbl, idx_rows)
    return out.reshape(b, l, EMBED)


# speedup vs baseline: 9.6566x; 9.6566x over previous
import functools

import jax
import jax.numpy as jnp
from jax import lax
from jax.experimental import pallas as pl
from jax.experimental.pallas import tpu as pltpu
from jax.experimental.pallas import tpu_sc as plsc

VOCAB = 100000
EMBED = 32
BATCH = 16384
SEQ = 200

NC, NS = 2, 16
NW = NC * NS
NROWS = BATCH * SEQ
IW = 128
NIDXROWS = NROWS // IW
IDXROWS_PER_W = NIDXROWS // NW
CH = 8
ITERS = IDXROWS_PER_W // CH


@functools.partial(
    pl.kernel,
    out_type=jax.ShapeDtypeStruct((NIDXROWS, IW, EMBED), jnp.float32),
    mesh=plsc.VectorSubcoreMesh(core_axis_name="c", subcore_axis_name="s"),
    scratch_types=[
        pltpu.VMEM((CH, IW), jnp.int32),
        pltpu.VMEM((CH, IW, EMBED), jnp.float32),
        pltpu.SemaphoreType.DMA,
    ],
    compiler_params=pltpu.CompilerParams(use_tc_tiling_on_sc=False),
)
def _gather_all(tbl_hbm, idx_hbm, out_hbm, idx_v, rows_v, sem):
    wid = lax.axis_index("s") * NC + lax.axis_index("c")
    row0 = wid * IDXROWS_PER_W

    def body(g, carry):
        r0 = row0 + g * CH
        pltpu.sync_copy(idx_hbm.at[pl.ds(r0, CH)], idx_v)
        copies = [
            pltpu.async_copy(tbl_hbm.at[idx_v.at[j]], rows_v.at[j], sem)
            for j in range(CH)
        ]
        for c in copies:
            c.wait()
        pltpu.sync_copy(rows_v, out_hbm.at[pl.ds(r0, CH)])
        return carry

    lax.fori_loop(0, ITERS, body, 0)


def kernel(inchi, table, start_var):
    b, l = inchi.shape
    tok = inchi[:, :-1].astype(jnp.int32)
    idx = jnp.concatenate(
        [jnp.full((b, 1), VOCAB, jnp.int32), tok], axis=1)
    idx_rows = idx.reshape(NIDXROWS, IW)
    tbl = jnp.concatenate([table, start_var], axis=0)
    out = _gather_all(tbl, idx_rows)
    return out.reshape(b, l, EMBED)
